# baseline (device time: 29404 ns/iter reference)
import functools

import jax
import jax.numpy as jnp
from jax import lax
from jax.experimental import pallas as pl
from jax.experimental.pallas import tpu as pltpu

N_DEV = 32
N_TOK = 512
D_MODEL = 256
N_EXP = 128
E_LOCAL = N_EXP // N_DEV
D_OUT = 512
ROWS = N_TOK // N_DEV


def kernel(x, router_W, route_idx, expert_W):
    def body(x_ref, rw_ref, idx_ref, ew_ref, out_ref,
             partial_ref, recv_ref, send_sems, recv_sems):
        my = lax.axis_index("i")

        barrier = pltpu.get_barrier_semaphore()
        for o in range(1, N_DEV):
            pl.semaphore_signal(
                barrier, inc=1,
                device_id=((my + o) % N_DEV,),
                device_id_type=pl.DeviceIdType.MESH,
            )
        pl.semaphore_wait(barrier, N_DEV - 1)

        xv = x_ref[:, :]
        scores = jnp.dot(xv, rw_ref[:, :], preferred_element_type=jnp.float32)
        lane = lax.broadcasted_iota(jnp.int32, (N_TOK, N_EXP), 1)
        id0 = idx_ref[:, 0:1]
        id1 = idx_ref[:, 1:2]
        s0 = jnp.sum(jnp.where(lane == id0, scores, 0.0), axis=1, keepdims=True)
        s1 = jnp.sum(jnp.where(lane == id1, scores, 0.0), axis=1, keepdims=True)
        w0 = 1.0 / (1.0 + jnp.exp(s1 - s0))
        w1 = 1.0 - w0

        acc = jnp.zeros((N_TOK, D_OUT), jnp.float32)
        for el in range(E_LOCAL):
            e = my * E_LOCAL + el
            coef = jnp.where(id0 == e, w0, 0.0) + jnp.where(id1 == e, w1, 0.0)
            xe = (coef * xv).astype(jnp.bfloat16)
            acc = acc + jnp.dot(
                xe, ew_ref[el, :, :].astype(jnp.bfloat16),
                preferred_element_type=jnp.float32,
            )
        partial_ref[:, :] = acc.astype(jnp.bfloat16)

        recv_ref[0, :, :] = partial_ref[pl.ds(my * ROWS, ROWS), :]

        rdmas = []
        for o in range(1, N_DEV):
            tgt = (my + o) % N_DEV
            rdma = pltpu.make_async_remote_copy(
                src_ref=partial_ref.at[pl.ds(tgt * ROWS, ROWS), :],
                dst_ref=recv_ref.at[o],
                send_sem=send_sems.at[o],
                recv_sem=recv_sems.at[o],
                device_id=(tgt,),
                device_id_type=pl.DeviceIdType.MESH,
            )
            rdma.start()
            rdmas.append(rdma)
        for rdma in rdmas:
            rdma.wait_send()
        for rdma in rdmas:
            rdma.wait_recv()

        out_ref[:, :] = jnp.sum(recv_ref[:, :, :].astype(jnp.float32), axis=0)

        @functools.partial(pl.run_scoped, sem=pltpu.SemaphoreType.REGULAR)
        def _(sem):
            for o in range(1, N_DEV):
                pl.semaphore_signal(
                    sem, inc=1,
                    device_id=((my + o) % N_DEV,),
                    device_id_type=pl.DeviceIdType.MESH,
                )
            pl.semaphore_wait(sem, N_DEV - 1)

    return pl.pallas_call(
        body,
        out_shape=jax.ShapeDtypeStruct((ROWS, D_OUT), jnp.float32),
        in_specs=[pl.BlockSpec(memory_space=pltpu.VMEM)] * 4,
        out_specs=pl.BlockSpec(memory_space=pltpu.VMEM),
        scratch_shapes=[
            pltpu.VMEM((N_TOK, D_OUT), jnp.bfloat16),
            pltpu.VMEM((N_DEV, ROWS, D_OUT), jnp.bfloat16),
            pltpu.SemaphoreType.DMA((N_DEV,)),
            pltpu.SemaphoreType.DMA((N_DEV,)),
        ],
        compiler_params=pltpu.CompilerParams(collective_id=0),
    )(x, router_W, route_idx, expert_W)


# device time: 18547 ns/iter; 1.5854x vs baseline; 1.5854x over previous
import jax
import jax.numpy as jnp
from jax import lax
from jax.experimental import pallas as pl
from jax.experimental.pallas import tpu as pltpu

N_DEV = 32
N_TOK = 512
D_MODEL = 256
N_EXP = 128
E_LOCAL = N_EXP // N_DEV
D_OUT = 512
ROWS = N_TOK // N_DEV


def kernel(x, router_W, route_idx, expert_W):
    def body(x_ref, rw_ref, idx_ref, ew_ref, out_ref,
             partial_ref, recv_ref, send_sems, recv_sems, credit_sems):
        my = lax.axis_index("i")

        barrier = pltpu.get_barrier_semaphore()
        pl.semaphore_signal(
            barrier, inc=1, device_id=(my,),
            device_id_type=pl.DeviceIdType.MESH,
        )
        pl.semaphore_wait(barrier, 1)

        for o in range(1, N_DEV):
            pl.semaphore_signal(
                credit_sems.at[o], inc=1,
                device_id=((my - o) % N_DEV,),
                device_id_type=pl.DeviceIdType.MESH,
            )

        xv = x_ref[:, :]
        scores = jnp.dot(
            xv.astype(jnp.bfloat16), rw_ref[:, :].astype(jnp.bfloat16),
            preferred_element_type=jnp.float32,
        )
        lane = lax.broadcasted_iota(jnp.int32, (N_TOK, N_EXP), 1)
        id0 = idx_ref[:, 0:1]
        id1 = idx_ref[:, 1:2]
        s0 = jnp.sum(jnp.where(lane == id0, scores, 0.0), axis=1, keepdims=True)
        s1 = jnp.sum(jnp.where(lane == id1, scores, 0.0), axis=1, keepdims=True)
        w0 = 1.0 / (1.0 + jnp.exp(s1 - s0))
        w1 = 1.0 - w0

        gated = []
        for el in range(E_LOCAL):
            e = my * E_LOCAL + el
            coef = jnp.where(id0 == e, w0, 0.0) + jnp.where(id1 == e, w1, 0.0)
            gated.append((coef * xv).astype(jnp.bfloat16))
        xg = jnp.concatenate(gated, axis=1)
        wg = ew_ref[:, :, :].astype(jnp.bfloat16).reshape(
            E_LOCAL * D_MODEL, D_OUT)

        BLK = 128

        def start_sends(lo, hi):
            for o in range(1, N_DEV):
                tgt = (my + o) % N_DEV

                @pl.when(jnp.logical_and(tgt * ROWS >= lo, tgt * ROWS < hi))
                def _():
                    pl.semaphore_wait(credit_sems.at[o], 1)
                    pltpu.make_async_remote_copy(
                        src_ref=partial_ref.at[pl.ds(tgt * ROWS, ROWS), :],
                        dst_ref=recv_ref.at[o],
                        send_sem=send_sems.at[o],
                        recv_sem=recv_sems.at[o],
                        device_id=(tgt,),
                        device_id_type=pl.DeviceIdType.MESH,
                    ).start()

        for b in range(N_TOK // BLK):
            lo = b * BLK
            partial_ref[lo:lo + BLK, :] = jnp.dot(
                xg[lo:lo + BLK, :], wg, preferred_element_type=jnp.float32
            ).astype(jnp.bfloat16)
            start_sends(lo, lo + BLK)

        recv_ref[0, :, :] = partial_ref[pl.ds(my * ROWS, ROWS), :]

        waiters = []
        for o in range(1, N_DEV):
            waiters.append(pltpu.make_async_remote_copy(
                src_ref=partial_ref.at[pl.ds(0, ROWS), :],
                dst_ref=recv_ref.at[o],
                send_sem=send_sems.at[o],
                recv_sem=recv_sems.at[o],
                device_id=(my,),
                device_id_type=pl.DeviceIdType.MESH,
            ))
        for w in waiters:
            w.wait_recv()
        out_ref[:, :] = jnp.sum(recv_ref[:, :, :].astype(jnp.float32), axis=0)
        for w in waiters:
            w.wait_send()

    return pl.pallas_call(
        body,
        out_shape=jax.ShapeDtypeStruct((ROWS, D_OUT), jnp.float32),
        in_specs=[pl.BlockSpec(memory_space=pltpu.VMEM)] * 4,
        out_specs=pl.BlockSpec(memory_space=pltpu.VMEM),
        scratch_shapes=[
            pltpu.VMEM((N_TOK, D_OUT), jnp.bfloat16),
            pltpu.VMEM((N_DEV, ROWS, D_OUT), jnp.bfloat16),
            pltpu.SemaphoreType.DMA((N_DEV,)),
            pltpu.SemaphoreType.DMA((N_DEV,)),
            pltpu.SemaphoreType.REGULAR((N_DEV,)),
        ],
        compiler_params=pltpu.CompilerParams(collective_id=0),
    )(x, router_W, route_idx, expert_W)
